# baseline (speedup 1.0000x reference)
import functools
import math

import jax
import jax.numpy as jnp
from jax import lax
from jax.experimental import pallas as pl
from jax.experimental.pallas import tpu as pltpu
from jax.experimental.pallas import tpu_sc as plsc

_NC = 2
_NS = 16
_NW = _NC * _NS


@functools.lru_cache(maxsize=None)
def _make_repack(V, D, rblk):
    mesh = plsc.VectorSubcoreMesh(core_axis_name="c", subcore_axis_name="s")
    nblk = V // rblk
    rem = V - nblk * rblk
    rem_a = (rem // 128) * 128
    rem_b = rem - rem_a
    per_w = -(-nblk // _NW)

    @functools.partial(
        pl.kernel,
        out_type=jax.ShapeDtypeStruct((V // 2, 2 * D), jnp.float32),
        mesh=mesh,
        scratch_types=[
            pltpu.VMEM((D, rblk), jnp.float32),
            pltpu.VMEM((rblk // 2, 2 * D), jnp.float32),
            pltpu.VMEM((D, rem_b), jnp.float32) if rem_b else None,
        ],
        compiler_params=pltpu.CompilerParams(use_tc_tiling_on_sc=True,
                                             needs_layout_passes=False),
    )
    def repack_fn(tt_hbm, tail_hbm, out_hbm, buf, obuf, tbuf):
        wid = lax.axis_index("s") * _NC + lax.axis_index("c")
        iota16 = lax.iota(jnp.int32, 16)
        cvecs = [iota16 + c0 * 16 for c0 in range(D // 16)]

        def do_block(r0, ncols):
            r0 = pl.multiple_of(r0, 128)
            p0 = pl.multiple_of(r0 // 2, 64)
            pltpu.sync_copy(tt_hbm.at[:, pl.ds(r0, ncols)],
                            buf.at[:, pl.ds(0, ncols)])

            def prow(p, c):
                for j in range(2):
                    rv = jnp.broadcast_to(2 * p + j, (16,))
                    for cg in range(D // 16):
                        vals = plsc.load_gather(buf, [cvecs[cg], rv])
                        obuf[p, pl.ds(j * D + cg * 16, 16)] = vals
                return c

            lax.fori_loop(0, ncols // 2, prow, 0)
            pltpu.sync_copy(obuf.at[pl.ds(0, ncols // 2)],
                            out_hbm.at[pl.ds(p0, ncols // 2)])

        def bloop(k, c):
            b = k * _NW + wid

            @pl.when(b < nblk)
            def _():
                do_block(b * rblk, rblk)

            return c

        lax.fori_loop(0, per_w, bloop, 0)
        if rem_a:
            @pl.when(wid == 0)
            def _():
                do_block(jnp.int32(nblk * rblk), rem_a)
        if rem_b:
            @pl.when(wid == 1)
            def _():
                pltpu.sync_copy(tail_hbm, tbuf)

                def prow_t(p, c):
                    for j in range(2):
                        rv = jnp.broadcast_to(2 * p + j, (16,))
                        for cg in range(D // 16):
                            vals = plsc.load_gather(tbuf, [cvecs[cg], rv])
                            obuf[p, pl.ds(j * D + cg * 16, 16)] = vals
                    return c

                lax.fori_loop(0, rem_b // 2, prow_t, 0)
                pltpu.sync_copy(obuf.at[pl.ds(0, rem_b // 2)],
                                out_hbm.at[pl.ds((V - rem_b) // 2,
                                                 rem_b // 2)])

    return repack_fn


@functools.lru_cache(maxsize=None)
def _make_sc_gather(V, n, L, D, scale):
    bpw = n // _NW
    assert bpw == 128 and D == 64
    mesh = plsc.VectorSubcoreMesh(core_axis_name="c", subcore_axis_name="s")
    RING = 4
    NOUT = 2
    DT = D // 8

    @functools.partial(
        pl.kernel,
        out_type=jax.ShapeDtypeStruct((L, DT, _NW, 8, bpw), jnp.float32),
        mesh=mesh,
        scratch_types=[
            pltpu.VMEM((L * D,), jnp.float32),
            pltpu.VMEM((L, bpw), jnp.int32),
            [pltpu.VMEM((bpw, D), jnp.float32) for _ in range(RING)],
            [pltpu.VMEM((DT, 8, bpw), jnp.float32) for _ in range(NOUT)],
            [pltpu.SemaphoreType.DMA for _ in range(RING)],
            [pltpu.SemaphoreType.DMA for _ in range(NOUT)],
        ],
        compiler_params=pltpu.CompilerParams(use_tc_tiling_on_sc=False, needs_layout_passes=False),
    )
    def sc_fn(table_hbm, xt_hbm, pe_hbm, out_hbm, pe_v, idx_v, rings, outs,
              gsems, ssems):
        wid = lax.axis_index("s") * _NC + lax.axis_index("c")
        pltpu.sync_copy(pe_hbm, pe_v)
        pltpu.sync_copy(xt_hbm.at[:, pl.ds(wid * bpw, bpw)], idx_v)

        def issue_gather(l, s):
            pltpu.async_copy(table_hbm.at[idx_v.at[l]], rings[s], gsems[s])

        def wait_gather(s):
            pltpu.make_async_copy(table_hbm.at[idx_v.at[0]], rings[s],
                                  gsems[s]).wait()

        def issue_store(l, o):
            pltpu.async_copy(outs[o], out_hbm.at[l, :, wid], ssems[o])

        def wait_store(o):
            pltpu.make_async_copy(outs[o], out_hbm.at[0, :, wid],
                                  ssems[o]).wait()

        for s in range(RING):
            issue_gather(jnp.int32(s), s)

        iota16 = lax.iota(jnp.int32, 16)
        brows = [iota16 + (bg * 16) for bg in range(bpw // 16)]

        nsteps = L // RING

        def step(j, carry):
            for s in range(RING):
                o = s % NOUT
                l = j * RING + s
                wait_gather(s)

                @pl.when(l >= NOUT)
                def _():
                    wait_store(o)

                def dloop(dc, c):
                    pe16 = pe_v[pl.ds(l * D + dc * 16, 16)]
                    for k in range(16):
                        d = dc * 16 + k
                        pes = pe16[k]
                        dcol = jnp.broadcast_to(d, (16,))
                        dt = dc * 2 + (k // 8)
                        for bg in range(bpw // 16):
                            vals = plsc.load_gather(rings[s],
                                                    [brows[bg], dcol])
                            outs[o][dt, k % 8, pl.ds(bg * 16, 16)] = (
                                vals * scale + pes)
                    return c

                lax.fori_loop(0, D // 16, dloop, 0)
                issue_store(l, o)
                nl = l + RING

                @pl.when(nl < L)
                def _():
                    issue_gather(nl, s)

            return carry

        lax.fori_loop(0, nsteps, step, 0)
        for o in range(NOUT):
            wait_store(o)

    return sc_fn


def _positional_encoding(embedding_dim, max_len=10000):
    position = jnp.arange(0, max_len, dtype=jnp.float32)[:, None]
    half = embedding_dim // 2
    div_term = jnp.exp(
        jnp.arange(0, half, dtype=jnp.float32) * -(math.log(10000.0) / (half - 1)))
    return jnp.concatenate(
        [jnp.sin(position * div_term), jnp.cos(position * div_term)], axis=1)


def kernel(x, timestep, table):
    n, L = x.shape
    V, D = table.shape
    scale = float(D ** 0.5)
    pe = _positional_encoding(D)
    pe_slice = lax.dynamic_slice_in_dim(pe, timestep, L, axis=0).reshape(-1)
    table_t = jnp.transpose(table)
    tail = lax.slice(table_t, (0, V - 64), (D, V))
    t128 = _make_repack(V, D, 896)(table_t, tail)
    table_lin = t128.reshape(V, D)
    x_t = jnp.transpose(x)
    out5 = _make_sc_gather(V, n, L, D, scale)(table_lin, x_t, pe_slice)
    return out5.transpose(2, 4, 0, 1, 3).reshape(n, L, D)


# speedup vs baseline: 1.1765x; 1.1765x over previous
import functools
import math

import jax
import jax.numpy as jnp
from jax import lax
from jax.experimental import pallas as pl
from jax.experimental.pallas import tpu as pltpu
from jax.experimental.pallas import tpu_sc as plsc

_NC = 2
_NS = 16
_NW = _NC * _NS


@functools.lru_cache(maxsize=None)
def _make_repack(V, D, rblk):
    mesh = plsc.VectorSubcoreMesh(core_axis_name="c", subcore_axis_name="s")
    nblk = V // rblk
    rem = V - nblk * rblk
    rem_a = (rem // 128) * 128
    rem_b = rem - rem_a
    per_w = -(-nblk // _NW)

    @functools.partial(
        pl.kernel,
        out_type=jax.ShapeDtypeStruct((V // 2, 2 * D), jnp.float32),
        mesh=mesh,
        scratch_types=[
            [pltpu.VMEM((D, rblk), jnp.float32) for _ in range(2)],
            [pltpu.VMEM((rblk // 2, 2 * D), jnp.float32) for _ in range(2)],
            pltpu.VMEM((D, rem_b), jnp.float32) if rem_b else None,
            [pltpu.SemaphoreType.DMA for _ in range(2)],
            [pltpu.SemaphoreType.DMA for _ in range(2)],
        ],
        compiler_params=pltpu.CompilerParams(use_tc_tiling_on_sc=True,
                                             needs_layout_passes=False),
    )
    def repack_fn(tt_hbm, tail_hbm, out_hbm, bufs, obufs, tbuf, rsems, wsems):
        wid = lax.axis_index("s") * _NC + lax.axis_index("c")
        iota16 = lax.iota(jnp.int32, 16)
        cvecs = [iota16 + c0 * 16 for c0 in range(D // 16)]

        def blk_r0(k):
            b = jnp.minimum(k * _NW + wid, nblk - 1)
            return pl.multiple_of(b * rblk, 128)

        def issue_read(k, h):
            pltpu.async_copy(tt_hbm.at[:, pl.ds(blk_r0(k), rblk)],
                             bufs[h], rsems[h])

        half_iota = lax.shift_right_logical(iota16, 1)
        odd64 = (iota16 & 1) * D

        def transpose_block(buf, obuf, ncols):
            def cloop(c, _):
                colv = odd64 + c

                def rloop(rr, __):
                    v = buf[c, pl.ds(rr * 16, 16)]
                    rowv = half_iota + rr * 8
                    plsc.store_scatter(obuf, [rowv, colv], v)
                    return __

                lax.fori_loop(0, ncols // 16, rloop, 0, unroll=4)
                return _

            lax.fori_loop(0, D, cloop, 0)

        nk2 = -(-per_w // 2)

        issue_read(jnp.int32(0), 0)
        issue_read(jnp.int32(1), 1)

        def kloop(k, c):
            for h in range(2):
                kk = 2 * k + h
                pltpu.make_async_copy(tt_hbm.at[:, pl.ds(0, rblk)],
                                      bufs[h], rsems[h]).wait()

                @pl.when(kk >= 2)
                def _():
                    pltpu.make_async_copy(
                        obufs[h], out_hbm.at[pl.ds(0, rblk // 2)],
                        wsems[h]).wait()

                transpose_block(bufs[h], obufs[h], rblk)
                p0 = pl.multiple_of(blk_r0(kk) // 2, 64)
                pltpu.async_copy(obufs[h],
                                 out_hbm.at[pl.ds(p0, rblk // 2)],
                                 wsems[h])

                @pl.when(kk + 2 < 2 * nk2)
                def _():
                    issue_read(kk + 2, h)

            return c

        lax.fori_loop(0, nk2, kloop, 0)
        for h in range(2):
            pltpu.make_async_copy(obufs[h],
                                  out_hbm.at[pl.ds(0, rblk // 2)],
                                  wsems[h]).wait()
        if rem_b:
            @pl.when(wid == 1)
            def _():
                pltpu.sync_copy(tail_hbm, tbuf)

                def prow_t(p, c):
                    for j in range(2):
                        rv = jnp.broadcast_to(2 * p + j, (16,))
                        for cg in range(D // 16):
                            vals = plsc.load_gather(tbuf, [cvecs[cg], rv])
                            obufs[0][p, pl.ds(j * D + cg * 16, 16)] = vals
                    return c

                lax.fori_loop(0, rem_b // 2, prow_t, 0)
                pltpu.sync_copy(obufs[0].at[pl.ds(0, rem_b // 2)],
                                out_hbm.at[pl.ds((V - rem_b) // 2,
                                                 rem_b // 2)])

    return repack_fn


@functools.lru_cache(maxsize=None)
def _make_sc_gather(V, n, L, D, scale):
    bpw = n // _NW
    assert bpw == 128 and D == 64
    mesh = plsc.VectorSubcoreMesh(core_axis_name="c", subcore_axis_name="s")
    RING = 4
    NOUT = 2
    DT = D // 8

    @functools.partial(
        pl.kernel,
        out_type=jax.ShapeDtypeStruct((L, DT, _NW, 8, bpw), jnp.float32),
        mesh=mesh,
        scratch_types=[
            [pltpu.VMEM((D * 16,), jnp.float32) for _ in range(RING)],
            pltpu.VMEM((L, bpw), jnp.int32),
            [pltpu.VMEM((bpw, D), jnp.float32) for _ in range(RING)],
            [pltpu.VMEM((DT, 8, bpw), jnp.float32) for _ in range(NOUT)],
            [pltpu.SemaphoreType.DMA for _ in range(RING)],
            [pltpu.SemaphoreType.DMA for _ in range(NOUT)],
        ],
        compiler_params=pltpu.CompilerParams(use_tc_tiling_on_sc=False, needs_layout_passes=False),
    )
    def sc_fn(table_hbm, xt_hbm, pe_hbm, out_hbm, pebs, idx_v, rings, outs,
              gsems, ssems):
        wid = lax.axis_index("s") * _NC + lax.axis_index("c")
        pltpu.sync_copy(xt_hbm.at[:, pl.ds(wid * bpw, bpw)], idx_v)

        def issue_gather(l, s):
            pltpu.async_copy(table_hbm.at[idx_v.at[l]], rings[s], gsems[s])
            pltpu.async_copy(pe_hbm.at[l], pebs[s], gsems[s])

        def wait_gather(s):
            pltpu.make_async_copy(table_hbm.at[idx_v.at[0]], rings[s],
                                  gsems[s]).wait()
            pltpu.make_async_copy(pe_hbm.at[0], pebs[s], gsems[s]).wait()

        def issue_store(l, o):
            pltpu.async_copy(outs[o], out_hbm.at[l, :, wid], ssems[o])

        def wait_store(o):
            pltpu.make_async_copy(outs[o], out_hbm.at[0, :, wid],
                                  ssems[o]).wait()

        for s in range(RING):
            issue_gather(jnp.int32(s), s)

        iota16 = lax.iota(jnp.int32, 16)
        brows = [iota16 + (bg * 16) for bg in range(bpw // 16)]

        nsteps = L // RING

        def step(j, carry):
            for s in range(RING):
                o = s % NOUT
                l = j * RING + s
                wait_gather(s)

                @pl.when(l >= NOUT)
                def _():
                    wait_store(o)

                def dloop(dc, c):
                    for k in range(8):
                        d = dc * 8 + k
                        pev = pebs[s][pl.ds(d * 16, 16)]
                        dcol = jnp.broadcast_to(d, (16,))
                        for bg in range(bpw // 16):
                            vals = plsc.load_gather(rings[s],
                                                    [brows[bg], dcol])
                            outs[o][dc, k, pl.ds(bg * 16, 16)] = (
                                vals * scale + pev)
                    return c

                lax.fori_loop(0, DT, dloop, 0)
                issue_store(l, o)
                nl = l + RING

                @pl.when(nl < L)
                def _():
                    issue_gather(nl, s)

            return carry

        lax.fori_loop(0, nsteps, step, 0)
        for o in range(NOUT):
            wait_store(o)

    return sc_fn


def _positional_encoding(embedding_dim, max_len=10000):
    position = jnp.arange(0, max_len, dtype=jnp.float32)[:, None]
    half = embedding_dim // 2
    div_term = jnp.exp(
        jnp.arange(0, half, dtype=jnp.float32) * -(math.log(10000.0) / (half - 1)))
    return jnp.concatenate(
        [jnp.sin(position * div_term), jnp.cos(position * div_term)], axis=1)


def kernel(x, timestep, table):
    n, L = x.shape
    V, D = table.shape
    scale = float(D ** 0.5)
    pe = _positional_encoding(D)
    pe_slice = lax.dynamic_slice_in_dim(pe, timestep, L, axis=0)
    pe_b = jnp.repeat(pe_slice[:, :, None], 16, axis=2).reshape(L, D * 16)
    table_t = jnp.transpose(table)
    tail = lax.slice(table_t, (0, V - 64), (D, V))
    t128 = _make_repack(V, D, 384)(table_t, tail)
    table_lin = t128.reshape(V, D)
    x_t = jnp.transpose(x)
    out5 = _make_sc_gather(V, n, L, D, scale)(table_lin, x_t, pe_b)
    return out5.transpose(2, 4, 0, 1, 3).reshape(n, L, D)


# speedup vs baseline: 2.8030x; 2.3824x over previous
import functools
import math

import jax
import jax.numpy as jnp
from jax import lax
from jax.experimental import pallas as pl
from jax.experimental.pallas import tpu as pltpu
from jax.experimental.pallas import tpu_sc as plsc

_NC = 2
_NS = 16
_NW = _NC * _NS


@functools.lru_cache(maxsize=None)
def _make_repack(V, D, rblk):
    mesh = plsc.VectorSubcoreMesh(core_axis_name="c", subcore_axis_name="s")
    nblk = V // rblk
    rem = V - nblk * rblk
    rem_a = (rem // 128) * 128
    rem_b = rem - rem_a
    per_w = -(-nblk // _NW)

    @functools.partial(
        pl.kernel,
        out_type=jax.ShapeDtypeStruct((V * D,), jnp.float32),
        mesh=mesh,
        scratch_types=[
            [pltpu.VMEM((D, rblk), jnp.float32) for _ in range(2)],
            [pltpu.VMEM((rblk * D,), jnp.float32) for _ in range(2)],
            pltpu.VMEM((D, rem_b), jnp.float32) if rem_b else None,
            [pltpu.SemaphoreType.DMA for _ in range(2)],
            [pltpu.SemaphoreType.DMA for _ in range(2)],
        ],
        compiler_params=pltpu.CompilerParams(use_tc_tiling_on_sc=True,
                                             needs_layout_passes=False),
    )
    def repack_fn(tt_hbm, tail_hbm, out_hbm, bufs, obufs, tbuf,
                  rsems, wsems):
        wid = lax.axis_index("s") * _NC + lax.axis_index("c")
        iota16 = lax.iota(jnp.int32, 16)
        cvecs = [iota16 + c0 * 16 for c0 in range(D // 16)]

        def blk_r0(k):
            b = jnp.minimum(k * _NW + wid, nblk - 1)
            return pl.multiple_of(b * rblk, 128)

        def issue_read(k, h):
            pltpu.async_copy(tt_hbm.at[:, pl.ds(blk_r0(k), rblk)],
                             bufs[h], rsems[h])

        iota64 = iota16 * D

        def transpose_block(buf, obuf, ncols):
            def rotloop(rot, _):
                diag = (iota16 + rot) & 15
                diag64 = iota64 + diag

                def rloop(rr, __):
                    rvec = iota16 + rr * 16
                    for cg in range(D // 16):
                        cvec = diag + cg * 16
                        v = plsc.load_gather(buf, [cvec, rvec])
                        fidx = diag64 + (rr * (16 * D) + cg * 16)
                        plsc.store_scatter(obuf, [fidx], v)
                    return __

                lax.fori_loop(0, ncols // 16, rloop, 0, unroll=2)
                return _

            lax.fori_loop(0, 16, rotloop, 0)

        nk2 = -(-per_w // 2)

        issue_read(jnp.int32(0), 0)
        issue_read(jnp.int32(1), 1)

        def kloop(k, c):
            for h in range(2):
                kk = 2 * k + h
                pltpu.make_async_copy(tt_hbm.at[:, pl.ds(0, rblk)],
                                      bufs[h], rsems[h]).wait()

                @pl.when(kk >= 2)
                def _():
                    pltpu.make_async_copy(
                        obufs[h], out_hbm.at[pl.ds(0, rblk * D)],
                        wsems[h]).wait()

                transpose_block(bufs[h], obufs[h], rblk)
                f0 = pl.multiple_of(blk_r0(kk) * D, 8)
                pltpu.async_copy(obufs[h],
                                 out_hbm.at[pl.ds(f0, rblk * D)],
                                 wsems[h])

                @pl.when(kk + 2 < 2 * nk2)
                def _():
                    issue_read(kk + 2, h)

            return c

        lax.fori_loop(0, nk2, kloop, 0)
        for h in range(2):
            pltpu.make_async_copy(obufs[h],
                                  out_hbm.at[pl.ds(0, rblk * D)],
                                  wsems[h]).wait()
        if rem_b:
            @pl.when(wid == 1)
            def _():
                pltpu.sync_copy(tail_hbm, tbuf)
                transpose_block(tbuf, obufs[0], rem_b)
                pltpu.sync_copy(obufs[0].at[pl.ds(0, rem_b * D)],
                                out_hbm.at[pl.ds((V - rem_b) * D,
                                                 rem_b * D)])

    return repack_fn


@functools.lru_cache(maxsize=None)
def _make_sc_gather(V, n, L, D, scale):
    bpw = n // _NW
    assert bpw == 128 and D == 64
    mesh = plsc.VectorSubcoreMesh(core_axis_name="c", subcore_axis_name="s")
    RING = 4
    NOUT = 2
    DT = D // 8

    @functools.partial(
        pl.kernel,
        out_type=jax.ShapeDtypeStruct((L, DT, _NW, 8, bpw), jnp.float32),
        mesh=mesh,
        scratch_types=[
            pltpu.VMEM((L * D,), jnp.float32),
            pltpu.VMEM((L, bpw), jnp.int32),
            [pltpu.VMEM((bpw, D), jnp.float32) for _ in range(RING)],
            [pltpu.VMEM((DT, 8, bpw), jnp.float32) for _ in range(NOUT)],
            [pltpu.SemaphoreType.DMA for _ in range(RING)],
            [pltpu.SemaphoreType.DMA for _ in range(NOUT)],
        ],
        compiler_params=pltpu.CompilerParams(use_tc_tiling_on_sc=False, needs_layout_passes=False),
    )
    def sc_fn(table_hbm, xt_hbm, pe_hbm, out_hbm, pe_v, idx_v, rings, outs,
              gsems, ssems):
        wid = lax.axis_index("s") * _NC + lax.axis_index("c")
        pltpu.sync_copy(pe_hbm, pe_v)
        pltpu.sync_copy(xt_hbm.at[:, pl.ds(wid * bpw, bpw)], idx_v)

        def issue_gather(l, s):
            pltpu.async_copy(table_hbm.at[idx_v.at[l]], rings[s], gsems[s])

        def wait_gather(s):
            pltpu.make_async_copy(table_hbm.at[idx_v.at[0]], rings[s],
                                  gsems[s]).wait()

        def issue_store(l, o):
            pltpu.async_copy(outs[o], out_hbm.at[l, :, wid], ssems[o])

        def wait_store(o):
            pltpu.make_async_copy(outs[o], out_hbm.at[0, :, wid],
                                  ssems[o]).wait()

        for s in range(RING):
            issue_gather(jnp.int32(s), s)

        iota16 = lax.iota(jnp.int32, 16)
        brows = [iota16 + (bg * 16) for bg in range(bpw // 16)]

        nsteps = L // RING

        def step(j, carry):
            for s in range(RING):
                o = s % NOUT
                l = j * RING + s
                wait_gather(s)

                @pl.when(l >= NOUT)
                def _():
                    wait_store(o)

                def rotloop(rot, c):
                    diag = (iota16 + rot) & 15
                    for dg in range(D // 16):
                        dvec = diag + (dg * 16)
                        dtv = lax.shift_right_logical(dvec, 3)
                        kv = dvec & 7
                        pev = plsc.load_gather(pe_v, [dvec + l * D])
                        for bg in range(bpw // 16):
                            vals = plsc.load_gather(rings[s],
                                                    [brows[bg], dvec])
                            plsc.store_scatter(outs[o], [dtv, kv, brows[bg]],
                                               vals * scale + pev)
                    return c

                lax.fori_loop(0, 16, rotloop, 0)
                issue_store(l, o)
                nl = l + RING

                @pl.when(nl < L)
                def _():
                    issue_gather(nl, s)

            return carry

        lax.fori_loop(0, nsteps, step, 0)
        for o in range(NOUT):
            wait_store(o)

    return sc_fn


def _positional_encoding(embedding_dim, max_len=10000):
    position = jnp.arange(0, max_len, dtype=jnp.float32)[:, None]
    half = embedding_dim // 2
    div_term = jnp.exp(
        jnp.arange(0, half, dtype=jnp.float32) * -(math.log(10000.0) / (half - 1)))
    return jnp.concatenate(
        [jnp.sin(position * div_term), jnp.cos(position * div_term)], axis=1)


def kernel(x, timestep, table):
    n, L = x.shape
    V, D = table.shape
    scale = float(D ** 0.5)
    pe = _positional_encoding(D)
    pe_slice = lax.dynamic_slice_in_dim(pe, timestep, L, axis=0)
    table_t = jnp.transpose(table)
    tail = lax.slice(table_t, (0, V - 64), (D, V))
    t128 = _make_repack(V, D, 384)(table_t, tail)
    table_lin = t128.reshape(V, D)
    x_t = jnp.transpose(x)
    out5 = _make_sc_gather(V, n, L, D, scale)(table_lin, x_t, pe_slice.reshape(-1))
    return out5.transpose(2, 4, 0, 1, 3).reshape(n, L, D)
